# baseline (device time: 107990 ns/iter reference)
import functools

import jax

try:
    jax.config.update("jax_compilation_cache_dir", "/tmp/jax_persist_cache")
    jax.config.update("jax_persistent_cache_min_compile_time_secs", 1.0)
except Exception:
    pass

import jax.numpy as jnp
from jax import lax
from jax.experimental import pallas as pl
from jax.experimental.pallas import tpu as pltpu

N_DEV = 8
NCHUNK = 512


def kernel(x, w_mat):
    m_all, mper = x.shape
    kdim, n = w_mat.shape
    assert m_all == N_DEV * mper == kdim
    n_chunks = n // NCHUNK

    def body(
        x_ref,
        w_ref,
        out_ref,
        xb_ref,
        abuf_ref,
        wf32_ref,
        wb_ref,
        send_sems,
        recv_sems,
        wsems,
        lsem,
    ):
        my = lax.axis_index("i")

        def wcopy(c, slot):
            return pltpu.make_async_copy(
                w_ref.at[:, pl.ds(c * NCHUNK, NCHUNK)],
                wf32_ref.at[slot],
                wsems.at[slot],
            )

        wcopy(0, 0).start()

        xb_ref[:, :] = x_ref[:, :].astype(jnp.bfloat16)

        barrier = pltpu.get_barrier_semaphore()
        for p in range(1, N_DEV):
            pl.semaphore_signal(
                barrier,
                inc=1,
                device_id=((my + p) % N_DEV,),
                device_id_type=pl.DeviceIdType.MESH,
            )
        pl.semaphore_wait(barrier, N_DEV - 1)

        sends = []
        for p in range(1, N_DEV):
            dst = (my + p) % N_DEV
            rdma = pltpu.make_async_remote_copy(
                src_ref=xb_ref.at[pl.ds(dst * mper, mper), :],
                dst_ref=abuf_ref.at[:, pl.ds(my * mper, mper)],
                send_sem=send_sems.at[p - 1],
                recv_sem=recv_sems.at[p - 1],
                device_id=(dst,),
                device_id_type=pl.DeviceIdType.MESH,
            )
            rdma.start()
            sends.append(rdma)

        local_cp = pltpu.make_async_copy(
            xb_ref.at[pl.ds(my * mper, mper), :],
            abuf_ref.at[:, pl.ds(my * mper, mper)],
            lsem,
        )
        local_cp.start()

        wcopy(1, 1).start()
        wcopy(0, 0).wait()
        wb_ref[0, :, :] = wf32_ref[0, :, :].astype(jnp.bfloat16)

        for p in range(1, N_DEV):
            recv = pltpu.make_async_remote_copy(
                src_ref=xb_ref.at[pl.ds(0, mper), :],
                dst_ref=abuf_ref.at[:, pl.ds(0, mper)],
                send_sem=send_sems.at[p - 1],
                recv_sem=recv_sems.at[p - 1],
                device_id=((my + p) % N_DEV,),
                device_id_type=pl.DeviceIdType.MESH,
            )
            recv.wait_recv()
        local_cp.wait()

        for c in range(n_chunks):
            slot = c % 2
            if c + 2 < n_chunks:
                wcopy(c + 2, slot).start()
            y = jnp.dot(
                abuf_ref[:, :], wb_ref[slot], preferred_element_type=jnp.float32
            )
            out_ref[:, pl.ds(c * NCHUNK, NCHUNK)] = jax.nn.gelu(
                y, approximate=True
            )
            if c + 1 < n_chunks:
                nslot = (c + 1) % 2
                wcopy(0, nslot).wait()
                wb_ref[nslot, :, :] = wf32_ref[nslot, :, :].astype(jnp.bfloat16)

        for rdma in sends:
            rdma.wait_send()

        @functools.partial(pl.run_scoped, exit_sem=pltpu.SemaphoreType.REGULAR)
        def _(exit_sem):
            for p in range(1, N_DEV):
                pl.semaphore_signal(
                    exit_sem,
                    inc=1,
                    device_id=((my + p) % N_DEV,),
                    device_id_type=pl.DeviceIdType.MESH,
                )
            pl.semaphore_wait(exit_sem, N_DEV - 1)

    return pl.pallas_call(
        body,
        out_shape=jax.ShapeDtypeStruct((mper, n), jnp.float32),
        in_specs=[
            pl.BlockSpec(memory_space=pltpu.MemorySpace.VMEM),
            pl.BlockSpec(memory_space=pltpu.MemorySpace.HBM),
        ],
        out_specs=pl.BlockSpec(memory_space=pltpu.MemorySpace.VMEM),
        scratch_shapes=[
            pltpu.VMEM((m_all, mper), jnp.bfloat16),
            pltpu.VMEM((mper, kdim), jnp.bfloat16),
            pltpu.VMEM((2, kdim, NCHUNK), jnp.float32),
            pltpu.VMEM((2, kdim, NCHUNK), jnp.bfloat16),
            pltpu.SemaphoreType.DMA((N_DEV - 1,)),
            pltpu.SemaphoreType.DMA((N_DEV - 1,)),
            pltpu.SemaphoreType.DMA((2,)),
            pltpu.SemaphoreType.DMA,
        ],
        compiler_params=pltpu.CompilerParams(
            collective_id=0,
            vmem_limit_bytes=100 * 1024 * 1024,
        ),
    )(x, w_mat)
